# baseline (speedup 1.0000x reference)
import functools

import jax
import jax.numpy as jnp
from jax import lax
from jax.experimental import pallas as pl
from jax.experimental.pallas import tpu as pltpu
from jax.experimental.pallas import tpu_sc as plsc

_EPS = 1e-5
_IDX_CHUNK = 128


def _prep_body(tidx_ref, depth_ref, dw_ref, db_ref, ttab_ref, tw_ref, tb_ref,
               out_ref):
    x = depth_ref[...]
    mu = jnp.mean(x, axis=-1, keepdims=True)
    xc = x - mu
    var = jnp.mean(xc * xc, axis=-1, keepdims=True)
    d = xc * lax.rsqrt(var + _EPS) * dw_ref[...] + db_ref[...]

    t = ttab_ref[pl.ds(tidx_ref[0], 1), :]
    tmu = jnp.mean(t, axis=-1, keepdims=True)
    tc = t - tmu
    tvar = jnp.mean(tc * tc, axis=-1, keepdims=True)
    te = tc * lax.rsqrt(tvar + _EPS) * tw_ref[...] + tb_ref[...]

    out_ref[...] = jnp.concatenate(
        [d, jnp.broadcast_to(te, d.shape)], axis=-1)


def _make_gather(num_rows_total, row_width, nc, ns):
    nw = nc * ns
    rows_per_w = num_rows_total // nw
    n_chunks = rows_per_w // _IDX_CHUNK
    mesh = plsc.VectorSubcoreMesh(core_axis_name="c", subcore_axis_name="s")

    @functools.partial(
        pl.kernel,
        mesh=mesh,
        out_type=jax.ShapeDtypeStruct(
            (nw * n_chunks, _IDX_CHUNK, row_width), jnp.float32),
        scratch_types=[
            pltpu.VMEM((n_chunks, _IDX_CHUNK), jnp.int32),
            pltpu.VMEM((n_chunks, _IDX_CHUNK, row_width), jnp.float32),
            pltpu.SemaphoreType.DMA,
        ],
    )
    def gather_k(ctable_hbm, idx_hbm, out_hbm, idx_v, rows_v, gsem):
        wid = lax.axis_index("s") * nc + lax.axis_index("c")
        base = wid * n_chunks
        pltpu.sync_copy(idx_hbm.at[pl.ds(base, n_chunks)], idx_v)
        gathers = [
            pltpu.async_copy(ctable_hbm.at[idx_v.at[j]], rows_v.at[j], gsem)
            for j in range(n_chunks)
        ]
        for g in gathers:
            g.wait()
        pltpu.sync_copy(rows_v, out_hbm.at[pl.ds(base, n_chunks)])

    return gather_k


def kernel(layer_indices, layer_type, depth_table, depth_ln_w, depth_ln_b,
           type_table, type_ln_w, type_ln_b):
    v, d = depth_table.shape
    b = layer_indices.shape[0]

    tidx = jnp.asarray(layer_type, jnp.int32).reshape(1)
    ctable = pl.pallas_call(
        _prep_body,
        in_specs=[
            pl.BlockSpec(memory_space=pltpu.SMEM),
            pl.BlockSpec(memory_space=pltpu.VMEM),
            pl.BlockSpec(memory_space=pltpu.VMEM),
            pl.BlockSpec(memory_space=pltpu.VMEM),
            pl.BlockSpec(memory_space=pltpu.VMEM),
            pl.BlockSpec(memory_space=pltpu.VMEM),
            pl.BlockSpec(memory_space=pltpu.VMEM),
        ],
        out_shape=jax.ShapeDtypeStruct((v, 2 * d), jnp.float32),
    )(tidx, depth_table,
      depth_ln_w.reshape(1, d), depth_ln_b.reshape(1, d),
      type_table, type_ln_w.reshape(1, d), type_ln_b.reshape(1, d))

    info = plsc.get_sparse_core_info()
    nc, ns = info.num_cores, info.num_subcores
    idx2d = layer_indices.astype(jnp.int32).reshape(-1, _IDX_CHUNK)
    out3d = _make_gather(b, 2 * d, nc, ns)(ctable, idx2d)
    return out3d.reshape(b, 2 * d)


# speedup vs baseline: 1.0527x; 1.0527x over previous
import functools

import jax
import jax.numpy as jnp
from jax import lax
from jax.experimental import pallas as pl
from jax.experimental.pallas import tpu as pltpu
from jax.experimental.pallas import tpu_sc as plsc

_EPS = 1e-5
_IDX_CHUNK = 128


def _prep_body(depth_ref, dw_ref, db_ref, trow_ref, tw_ref, tb_ref, out_ref):
    x = depth_ref[...]
    mu = jnp.mean(x, axis=-1, keepdims=True)
    xc = x - mu
    var = jnp.mean(xc * xc, axis=-1, keepdims=True)
    d = xc * lax.rsqrt(var + _EPS) * dw_ref[...] + db_ref[...]

    t = trow_ref[...]
    tmu = jnp.mean(t, axis=-1, keepdims=True)
    tc = t - tmu
    tvar = jnp.mean(tc * tc, axis=-1, keepdims=True)
    te = tc * lax.rsqrt(tvar + _EPS) * tw_ref[...] + tb_ref[...]

    out_ref[...] = jnp.concatenate(
        [d, jnp.broadcast_to(te, d.shape)], axis=-1)


def _make_gather(num_rows_total, row_width, nc, ns):
    nw = nc * ns
    rows_per_w = num_rows_total // nw
    n_chunks = rows_per_w // _IDX_CHUNK
    mesh = plsc.VectorSubcoreMesh(core_axis_name="c", subcore_axis_name="s")

    @functools.partial(
        pl.kernel,
        mesh=mesh,
        out_type=jax.ShapeDtypeStruct(
            (nw * n_chunks, _IDX_CHUNK, row_width), jnp.float32),
        scratch_types=[
            pltpu.VMEM((n_chunks, _IDX_CHUNK), jnp.int32),
            pltpu.VMEM((n_chunks, _IDX_CHUNK, row_width), jnp.float32),
            pltpu.SemaphoreType.DMA,
        ],
    )
    def gather_k(ctable_hbm, idx_hbm, out_hbm, idx_v, rows_v, gsem):
        wid = lax.axis_index("s") * nc + lax.axis_index("c")
        base = wid * n_chunks
        pltpu.sync_copy(idx_hbm.at[pl.ds(base, n_chunks)], idx_v)
        gathers = [
            pltpu.async_copy(ctable_hbm.at[idx_v.at[j]], rows_v.at[j], gsem)
            for j in range(n_chunks)
        ]
        for g in gathers:
            g.wait()
        pltpu.sync_copy(rows_v, out_hbm.at[pl.ds(base, n_chunks)])

    return gather_k


def kernel(layer_indices, layer_type, depth_table, depth_ln_w, depth_ln_b,
           type_table, type_ln_w, type_ln_b):
    v, d = depth_table.shape
    b = layer_indices.shape[0]

    trow = lax.dynamic_slice_in_dim(
        type_table, jnp.asarray(layer_type, jnp.int32), 1, axis=0)
    ctable = pl.pallas_call(
        _prep_body,
        out_shape=jax.ShapeDtypeStruct((v, 2 * d), jnp.float32),
    )(depth_table,
      depth_ln_w.reshape(1, d), depth_ln_b.reshape(1, d),
      trow, type_ln_w.reshape(1, d), type_ln_b.reshape(1, d))

    info = plsc.get_sparse_core_info()
    nc, ns = info.num_cores, info.num_subcores
    idx2d = layer_indices.astype(jnp.int32).reshape(-1, _IDX_CHUNK)
    out3d = _make_gather(b, 2 * d, nc, ns)(ctable, idx2d)
    return out3d.reshape(b, 2 * d)


# speedup vs baseline: 1.1302x; 1.0737x over previous
import functools

import jax
import jax.numpy as jnp
from jax import lax
from jax.experimental import pallas as pl
from jax.experimental.pallas import tpu as pltpu
from jax.experimental.pallas import tpu_sc as plsc

_EPS = 1e-5
_IDX_CHUNK = 128


def _prep_body(depth_ref, dw_ref, db_ref, trow_ref, tw_ref, tb_ref, out_ref):
    x = depth_ref[...]
    mu = jnp.mean(x, axis=-1, keepdims=True)
    xc = x - mu
    var = jnp.mean(xc * xc, axis=-1, keepdims=True)
    d = xc * lax.rsqrt(var + _EPS) * dw_ref[...] + db_ref[...]

    t = trow_ref[...]
    tmu = jnp.mean(t, axis=-1, keepdims=True)
    tc = t - tmu
    tvar = jnp.mean(tc * tc, axis=-1, keepdims=True)
    te = tc * lax.rsqrt(tvar + _EPS) * tw_ref[...] + tb_ref[...]

    v = depth_ref.shape[0]
    out_ref[pl.ds(0, v), :] = jnp.concatenate(
        [d, jnp.broadcast_to(te, d.shape)], axis=-1)


def _make_gather(num_rows_total, table_rows_pad, row_width, nc, ns):
    nw = nc * ns
    rows_per_w = num_rows_total // nw
    n_chunks = rows_per_w // _IDX_CHUNK
    stage_rows = table_rows_pad // ns
    mesh = plsc.VectorSubcoreMesh(core_axis_name="c", subcore_axis_name="s")

    @functools.partial(
        pl.kernel,
        mesh=mesh,
        out_type=jax.ShapeDtypeStruct(
            (nw * n_chunks, _IDX_CHUNK, row_width), jnp.float32),
        scratch_types=[
            pltpu.VMEM_SHARED((table_rows_pad, row_width), jnp.float32),
            pltpu.VMEM((n_chunks, _IDX_CHUNK), jnp.int32),
            pltpu.VMEM((n_chunks, _IDX_CHUNK, row_width), jnp.float32),
            pltpu.SemaphoreType.DMA,
        ],
    )
    def gather_k(ctable_hbm, idx_hbm, out_hbm, table_sp, idx_v, rows_v, gsem):
        sid = lax.axis_index("s")
        wid = sid * nc + lax.axis_index("c")
        base = wid * n_chunks
        pltpu.sync_copy(ctable_hbm.at[pl.ds(sid * stage_rows, stage_rows)],
                        table_sp.at[pl.ds(sid * stage_rows, stage_rows)])
        pltpu.sync_copy(idx_hbm.at[pl.ds(base, n_chunks)], idx_v)
        plsc.subcore_barrier()
        gathers = [
            pltpu.async_copy(table_sp.at[idx_v.at[j]], rows_v.at[j], gsem)
            for j in range(n_chunks)
        ]
        for g in gathers:
            g.wait()
        pltpu.sync_copy(rows_v, out_hbm.at[pl.ds(base, n_chunks)])

    return gather_k


def kernel(layer_indices, layer_type, depth_table, depth_ln_w, depth_ln_b,
           type_table, type_ln_w, type_ln_b):
    v, d = depth_table.shape
    b = layer_indices.shape[0]
    v_pad = ((v + 127) // 128) * 128

    trow = lax.dynamic_slice_in_dim(
        type_table, jnp.asarray(layer_type, jnp.int32), 1, axis=0)
    ctable = pl.pallas_call(
        _prep_body,
        out_shape=jax.ShapeDtypeStruct((v_pad, 2 * d), jnp.float32),
    )(depth_table,
      depth_ln_w.reshape(1, d), depth_ln_b.reshape(1, d),
      trow, type_ln_w.reshape(1, d), type_ln_b.reshape(1, d))

    info = plsc.get_sparse_core_info()
    nc, ns = info.num_cores, info.num_subcores
    idx2d = layer_indices.astype(jnp.int32).reshape(-1, _IDX_CHUNK)
    out3d = _make_gather(b, v_pad, 2 * d, nc, ns)(ctable, idx2d)
    return out3d.reshape(b, 2 * d)


# speedup vs baseline: 1.1605x; 1.0268x over previous
import functools

import jax
import jax.numpy as jnp
from jax import lax
from jax.experimental import pallas as pl
from jax.experimental.pallas import tpu as pltpu
from jax.experimental.pallas import tpu_sc as plsc

_EPS = 1e-5
_IDX_CHUNK = 128


def _prep_body(depth_ref, dw_ref, db_ref, trow_ref, tw_ref, tb_ref, out_ref):
    x = depth_ref[...]
    mu = jnp.mean(x, axis=-1, keepdims=True)
    xc = x - mu
    var = jnp.mean(xc * xc, axis=-1, keepdims=True)
    d = xc * lax.rsqrt(var + _EPS) * dw_ref[...] + db_ref[...]

    t = trow_ref[...]
    tmu = jnp.mean(t, axis=-1, keepdims=True)
    tc = t - tmu
    tvar = jnp.mean(tc * tc, axis=-1, keepdims=True)
    te = tc * lax.rsqrt(tvar + _EPS) * tw_ref[...] + tb_ref[...]

    v = depth_ref.shape[0]
    out_ref[pl.ds(0, v), :] = jnp.concatenate(
        [d, jnp.broadcast_to(te, d.shape)], axis=-1)


def _make_gather(num_rows_total, table_rows_pad, row_width, nc, ns):
    nw = nc * ns
    rows_per_w = num_rows_total // nw
    n_chunks = rows_per_w // _IDX_CHUNK
    stage_rows = table_rows_pad // ns
    mesh = plsc.VectorSubcoreMesh(core_axis_name="c", subcore_axis_name="s")

    @functools.partial(
        pl.kernel,
        mesh=mesh,
        out_type=jax.ShapeDtypeStruct(
            (nw * n_chunks, _IDX_CHUNK, row_width), jnp.float32),
        scratch_types=[
            pltpu.VMEM_SHARED((table_rows_pad, row_width), jnp.float32),
            pltpu.VMEM((n_chunks, _IDX_CHUNK), jnp.int32),
            pltpu.VMEM((n_chunks, _IDX_CHUNK, row_width), jnp.float32),
            pltpu.SemaphoreType.DMA((n_chunks,)),
            pltpu.SemaphoreType.DMA,
        ],
    )
    def gather_k(ctable_hbm, idx_hbm, out_hbm, table_sp, idx_v, rows_v,
                 gsem, ssem):
        sid = lax.axis_index("s")
        wid = sid * nc + lax.axis_index("c")
        base = wid * n_chunks
        pltpu.sync_copy(ctable_hbm.at[pl.ds(sid * stage_rows, stage_rows)],
                        table_sp.at[pl.ds(sid * stage_rows, stage_rows)])
        pltpu.sync_copy(idx_hbm.at[pl.ds(base, n_chunks)], idx_v)
        plsc.subcore_barrier()
        gathers = [
            pltpu.async_copy(table_sp.at[idx_v.at[j]], rows_v.at[j],
                             gsem.at[j])
            for j in range(n_chunks)
        ]
        stores = []
        for j in range(n_chunks):
            gathers[j].wait()
            stores.append(pltpu.async_copy(
                rows_v.at[j], out_hbm.at[base + j], ssem))
        for st in stores:
            st.wait()

    return gather_k


def kernel(layer_indices, layer_type, depth_table, depth_ln_w, depth_ln_b,
           type_table, type_ln_w, type_ln_b):
    v, d = depth_table.shape
    b = layer_indices.shape[0]
    v_pad = ((v + 127) // 128) * 128

    trow = lax.dynamic_slice_in_dim(
        type_table, jnp.asarray(layer_type, jnp.int32), 1, axis=0)
    ctable = pl.pallas_call(
        _prep_body,
        out_shape=jax.ShapeDtypeStruct((v_pad, 2 * d), jnp.float32),
    )(depth_table,
      depth_ln_w.reshape(1, d), depth_ln_b.reshape(1, d),
      trow, type_ln_w.reshape(1, d), type_ln_b.reshape(1, d))

    info = plsc.get_sparse_core_info()
    nc, ns = info.num_cores, info.num_subcores
    idx2d = layer_indices.astype(jnp.int32).reshape(-1, _IDX_CHUNK)
    out3d = _make_gather(b, v_pad, 2 * d, nc, ns)(ctable, idx2d)
    return out3d.reshape(b, 2 * d)
